# baseline (speedup 1.0000x reference)
import functools

import jax
import jax.numpy as jnp
from jax.experimental import pallas as pl
from jax.experimental.pallas import tpu as pltpu


def _embed_kernel(x_ref, w_ref, add_ref, out_ref):
    x = x_ref[...].astype(jnp.bfloat16)
    w = w_ref[...].astype(jnp.bfloat16)
    acc = jnp.dot(x, w, preferred_element_type=jnp.float32)
    out_ref[...] = acc + add_ref[...]


@functools.partial(jax.jit, static_argnames=("tm", "tn", "interpret"))
def _embed(seq2d, W_tok, addend, tm=896, tn=2048, interpret=False):
    M, C = seq2d.shape
    D = W_tok.shape[1]
    grid = (D // tn, M // tm)
    out = pl.pallas_call(
        _embed_kernel,
        grid=grid,
        in_specs=[
            pl.BlockSpec((tm, C), lambda j, i: (i, 0)),
            pl.BlockSpec((C, tn), lambda j, i: (0, j)),
            pl.BlockSpec((tm, tn), lambda j, i: (0, j)),
        ],
        out_specs=pl.BlockSpec((tm, tn), lambda j, i: (i, j)),
        out_shape=jax.ShapeDtypeStruct((M, D), jnp.float32),
        compiler_params=pltpu.CompilerParams(
            dimension_semantics=("arbitrary", "arbitrary"),
        ),
        interpret=interpret,
    )(seq2d, W_tok, addend)
    return out


def kernel(sequence, W_tok, b_tok, pos_table):
    B, L, C = sequence.shape
    D = W_tok.shape[1]
    tm = 128 * L
    seq2d = sequence.reshape(B * L, C)
    addend = jnp.tile(pos_table + b_tok[None, :], (tm // L, 1))
    out = _embed(seq2d, W_tok, addend, tm=tm, tn=D)
    x = out.reshape(B, L, D)
    mask = jnp.ones((B, L), dtype=bool)
    return (x, mask)


# speedup vs baseline: 1.4295x; 1.4295x over previous
import functools

import jax
import jax.numpy as jnp
from jax.experimental import pallas as pl
from jax.experimental.pallas import tpu as pltpu


def _embed_kernel(x_ref, w_ref, padd_ref, out_ref):
    l_dim = x_ref.shape[1]
    w = w_ref[...]
    for l in range(l_dim):
        acc = jnp.dot(x_ref[:, l, :], w, preferred_element_type=jnp.float32)
        out_ref[:, l, :] = acc + padd_ref[l, :][None, :]


@functools.partial(jax.jit, static_argnames=("tb", "tn", "interpret"))
def _embed(sequence, W_tok, padd, tb=256, tn=1024, interpret=False):
    B, L, C = sequence.shape
    D = W_tok.shape[1]
    grid = (B // tb, D // tn)
    out = pl.pallas_call(
        _embed_kernel,
        grid=grid,
        in_specs=[
            pl.BlockSpec((tb, L, C), lambda i, j: (i, 0, 0)),
            pl.BlockSpec((C, tn), lambda i, j: (0, j)),
            pl.BlockSpec((L, tn), lambda i, j: (0, j)),
        ],
        out_specs=pl.BlockSpec((tb, L, tn), lambda i, j: (i, 0, j)),
        out_shape=jax.ShapeDtypeStruct((B, L, D), jnp.float32),
        compiler_params=pltpu.CompilerParams(
            dimension_semantics=("arbitrary", "arbitrary"),
        ),
        interpret=interpret,
    )(sequence, W_tok, padd)
    return out


def kernel(sequence, W_tok, b_tok, pos_table):
    B, L, C = sequence.shape
    padd = pos_table + b_tok[None, :]
    x = _embed(sequence, W_tok, padd)
    mask = jnp.ones((B, L), dtype=bool)
    return (x, mask)


# speedup vs baseline: 5.0561x; 3.5370x over previous
import functools

import jax
import jax.numpy as jnp
from jax.experimental import pallas as pl
from jax.experimental.pallas import tpu as pltpu


def _embed_kernel(x_ref, w_ref, padd_ref, out_ref):
    x = x_ref[0]
    acc = jnp.dot(x, w_ref[...], preferred_element_type=jnp.float32)
    out_ref[0] = acc + padd_ref[0]


@functools.partial(jax.jit, static_argnames=("tb", "tn", "interpret"))
def _embed(seq_t, W_tok, padd, tb=512, tn=2048, interpret=False):
    L, B, C = seq_t.shape
    D = W_tok.shape[1]
    grid = (L, B // tb, D // tn)
    out = pl.pallas_call(
        _embed_kernel,
        grid=grid,
        in_specs=[
            pl.BlockSpec((1, tb, C), lambda l, i, j: (l, i, 0)),
            pl.BlockSpec((C, tn), lambda l, i, j: (0, j)),
            pl.BlockSpec((1, 1, tn), lambda l, i, j: (l, 0, j)),
        ],
        out_specs=pl.BlockSpec((1, tb, tn), lambda l, i, j: (l, i, j)),
        out_shape=jax.ShapeDtypeStruct((L, B, D), jnp.float32),
        compiler_params=pltpu.CompilerParams(
            dimension_semantics=("arbitrary", "arbitrary", "arbitrary"),
        ),
        interpret=interpret,
    )(seq_t, W_tok, padd)
    return out


def kernel(sequence, W_tok, b_tok, pos_table):
    B, L, C = sequence.shape
    D = W_tok.shape[1]
    padd = (pos_table + b_tok[None, :]).reshape(L, 1, D)
    seq_t = jnp.transpose(sequence, (1, 0, 2))
    out_t = _embed(seq_t, W_tok, padd)
    x = jnp.transpose(out_t, (1, 0, 2))
    mask = jnp.ones((B, L), dtype=bool)
    return (x, mask)


# speedup vs baseline: 5.0609x; 1.0009x over previous
import functools

import jax
import jax.numpy as jnp
from jax.experimental import pallas as pl
from jax.experimental.pallas import tpu as pltpu


def _embed_kernel(x_ref, w_ref, padd_ref, out_ref):
    x = x_ref[0].astype(jnp.bfloat16)
    w = w_ref[...].astype(jnp.bfloat16)
    acc = jnp.dot(x, w, preferred_element_type=jnp.float32)
    out_ref[0] = acc + padd_ref[0]


@functools.partial(jax.jit, static_argnames=("tb", "tn", "interpret"))
def _embed(seq_t, W_tok, padd, tb=512, tn=2048, interpret=False):
    L, B, C = seq_t.shape
    D = W_tok.shape[1]
    grid = (L, B // tb, D // tn)
    out = pl.pallas_call(
        _embed_kernel,
        grid=grid,
        in_specs=[
            pl.BlockSpec((1, tb, C), lambda l, i, j: (l, i, 0)),
            pl.BlockSpec((C, tn), lambda l, i, j: (0, j)),
            pl.BlockSpec((1, 1, tn), lambda l, i, j: (l, 0, j)),
        ],
        out_specs=pl.BlockSpec((1, tb, tn), lambda l, i, j: (l, i, j)),
        out_shape=jax.ShapeDtypeStruct((L, B, D), jnp.float32),
        compiler_params=pltpu.CompilerParams(
            dimension_semantics=("arbitrary", "arbitrary", "arbitrary"),
        ),
        interpret=interpret,
    )(seq_t, W_tok, padd)
    return out


def kernel(sequence, W_tok, b_tok, pos_table):
    B, L, C = sequence.shape
    D = W_tok.shape[1]
    padd = (pos_table + b_tok[None, :]).reshape(L, 1, D)
    seq_t = jnp.transpose(sequence, (1, 0, 2))
    out_t = _embed(seq_t, W_tok, padd)
    x = jnp.transpose(out_t, (1, 0, 2))
    mask = jnp.ones((B, L), dtype=bool)
    return (x, mask)


# speedup vs baseline: 5.0672x; 1.0012x over previous
import functools

import jax
import jax.numpy as jnp
from jax.experimental import pallas as pl
from jax.experimental.pallas import tpu as pltpu


def _embed_kernel(x_ref, w_ref, padd_ref, out_ref):
    x = x_ref[0].astype(jnp.bfloat16)
    w = w_ref[...].astype(jnp.bfloat16)
    acc = jnp.dot(x, w, preferred_element_type=jnp.float32)
    out_ref[0] = acc + padd_ref[0]


@functools.partial(jax.jit, static_argnames=("tb", "tn", "interpret"))
def _embed(seq_t, W_tok, padd, tb=512, tn=2048, interpret=False):
    L, B, C = seq_t.shape
    D = W_tok.shape[1]
    grid = (L, B // tb, D // tn)
    out = pl.pallas_call(
        _embed_kernel,
        grid=grid,
        in_specs=[
            pl.BlockSpec((1, tb, C), lambda l, i, j: (l, i, 0)),
            pl.BlockSpec((C, tn), lambda l, i, j: (0, j)),
            pl.BlockSpec((1, 1, tn), lambda l, i, j: (l, 0, j)),
        ],
        out_specs=pl.BlockSpec((1, tb, tn), lambda l, i, j: (l, i, j)),
        out_shape=jax.ShapeDtypeStruct((L, B, D), jnp.float32),
        compiler_params=pltpu.CompilerParams(
            dimension_semantics=("parallel", "parallel", "parallel"),
        ),
        interpret=interpret,
    )(seq_t, W_tok, padd)
    return out


def kernel(sequence, W_tok, b_tok, pos_table):
    B, L, C = sequence.shape
    D = W_tok.shape[1]
    padd = (pos_table + b_tok[None, :]).reshape(L, 1, D)
    seq_t = jnp.transpose(sequence, (1, 0, 2))
    out_t = _embed(seq_t, W_tok, padd)
    x = jnp.transpose(out_t, (1, 0, 2))
    mask = jnp.ones((B, L), dtype=bool)
    return (x, mask)


# speedup vs baseline: 5.9047x; 1.1653x over previous
import functools

import jax
import jax.numpy as jnp
from jax.experimental import pallas as pl
from jax.experimental.pallas import tpu as pltpu


def _embed_kernel(x_ref, w_ref, padd_ref, out_ref):
    x = x_ref[0].astype(jnp.bfloat16)
    w = w_ref[...].astype(jnp.bfloat16)
    acc = jnp.dot(x, w, preferred_element_type=jnp.float32)
    out_ref[0] = acc + padd_ref[0]


@functools.partial(jax.jit, static_argnames=("tb", "tn", "interpret"))
def _embed(seq_t, W_tok, padd, tb=1024, tn=2048, interpret=False):
    L, B, C = seq_t.shape
    D = W_tok.shape[1]
    grid = (L, B // tb, D // tn)
    out = pl.pallas_call(
        _embed_kernel,
        grid=grid,
        in_specs=[
            pl.BlockSpec((1, tb, C), lambda l, i, j: (l, i, 0)),
            pl.BlockSpec((C, tn), lambda l, i, j: (0, j)),
            pl.BlockSpec((1, 1, tn), lambda l, i, j: (l, 0, j)),
        ],
        out_specs=pl.BlockSpec((1, tb, tn), lambda l, i, j: (l, i, j)),
        out_shape=jax.ShapeDtypeStruct((L, B, D), jnp.float32),
        compiler_params=pltpu.CompilerParams(
            dimension_semantics=("parallel", "parallel", "parallel"),
        ),
        interpret=interpret,
    )(seq_t, W_tok, padd)
    return out


def kernel(sequence, W_tok, b_tok, pos_table):
    B, L, C = sequence.shape
    D = W_tok.shape[1]
    padd = (pos_table + b_tok[None, :]).reshape(L, 1, D)
    seq_t = jnp.transpose(sequence, (1, 0, 2))
    out_t = _embed(seq_t, W_tok, padd)
    x = jnp.transpose(out_t, (1, 0, 2))
    mask = jnp.ones((B, L), dtype=bool)
    return (x, mask)


# speedup vs baseline: 6.0348x; 1.0220x over previous
import functools

import jax
import jax.numpy as jnp
from jax.experimental import pallas as pl
from jax.experimental.pallas import tpu as pltpu


def _embed_kernel(x_ref, w_ref, padd_ref, out_ref):
    x = x_ref[0].astype(jnp.bfloat16)
    w = w_ref[...].astype(jnp.bfloat16)
    acc = jnp.dot(x, w, preferred_element_type=jnp.float32)
    out_ref[0] = acc + padd_ref[0]


@functools.partial(jax.jit, static_argnames=("tb", "tn", "interpret"))
def _embed(seq_t, W_tok, padd, tb=2048, tn=2048, interpret=False):
    L, B, C = seq_t.shape
    D = W_tok.shape[1]
    grid = (L, B // tb, D // tn)
    out = pl.pallas_call(
        _embed_kernel,
        grid=grid,
        in_specs=[
            pl.BlockSpec((1, tb, C), lambda l, i, j: (l, i, 0)),
            pl.BlockSpec((C, tn), lambda l, i, j: (0, j)),
            pl.BlockSpec((1, 1, tn), lambda l, i, j: (l, 0, j)),
        ],
        out_specs=pl.BlockSpec((1, tb, tn), lambda l, i, j: (l, i, j)),
        out_shape=jax.ShapeDtypeStruct((L, B, D), jnp.float32),
        compiler_params=pltpu.CompilerParams(
            dimension_semantics=("parallel", "parallel", "parallel"),
        ),
        interpret=interpret,
    )(seq_t, W_tok, padd)
    return out


def kernel(sequence, W_tok, b_tok, pos_table):
    B, L, C = sequence.shape
    D = W_tok.shape[1]
    padd = (pos_table + b_tok[None, :]).reshape(L, 1, D)
    seq_t = jnp.transpose(sequence, (1, 0, 2))
    out_t = _embed(seq_t, W_tok, padd)
    x = jnp.transpose(out_t, (1, 0, 2))
    mask = jnp.ones((B, L), dtype=bool)
    return (x, mask)
